# baseline (speedup 1.0000x reference)
import functools

import jax
import jax.numpy as jnp
from jax import lax
from jax.experimental import pallas as pl
from jax.experimental.pallas import tpu as pltpu
from jax.experimental.pallas import tpu_sc as plsc

D = 128
L = 16
E = 80
SCH = 2000
NBUF = 4


@functools.lru_cache(maxsize=None)
def _make_decoder(N, B):
    info = plsc.get_sparse_core_info()
    NC, NS = info.num_cores, info.num_subcores
    NW = NC * NS
    assert B % NW == 0
    per_w = B // NW
    assert per_w % SCH == 0 and SCH % E == 0
    n_sch = per_w // SCH
    n_chunks = SCH // E
    mesh = plsc.VectorSubcoreMesh(core_axis_name="c", subcore_axis_name="s")

    @functools.partial(
        pl.kernel,
        out_type=jax.ShapeDtypeStruct((B,), jnp.float32),
        mesh=mesh,
        compiler_params=pltpu.CompilerParams(needs_layout_passes=False),
        scratch_types=[
            pltpu.VMEM((SCH,), jnp.int32),
            pltpu.VMEM((SCH,), jnp.int32),
            pltpu.VMEM((NBUF * E, D), jnp.float32),
            pltpu.VMEM((NBUF * E, D), jnp.float32),
            pltpu.VMEM((SCH,), jnp.float32),
            pltpu.VMEM((L, L), jnp.float32),
            pltpu.SemaphoreType.DMA((NBUF,)),
            pltpu.SemaphoreType.DMA((NBUF,)),
        ],
    )
    def decode(z_hbm, src_hbm, dst_hbm, out_hbm,
               sidx, didx, srows, drows, och, pbuf, sem_s, sem_d):
        wid = lax.axis_index("s") * NC + lax.axis_index("c")
        wbase = wid * per_w

        def start_gathers(c, buf):
            pltpu.async_copy(
                z_hbm.at[sidx.at[pl.ds(c * E, E)]],
                srows.at[pl.ds(buf * E, E)], sem_s.at[buf])
            pltpu.async_copy(
                z_hbm.at[didx.at[pl.ds(c * E, E)]],
                drows.at[pl.ds(buf * E, E)], sem_d.at[buf])

        def wait_gathers(c, buf):
            pltpu.make_async_copy(
                z_hbm.at[sidx.at[pl.ds(c * E, E)]],
                srows.at[pl.ds(buf * E, E)], sem_s.at[buf]).wait()
            pltpu.make_async_copy(
                z_hbm.at[didx.at[pl.ds(c * E, E)]],
                drows.at[pl.ds(buf * E, E)], sem_d.at[buf]).wait()

        def sch_body(j, carry0):
            scbase = wbase + j * SCH
            pltpu.sync_copy(src_hbm.at[pl.ds(scbase, SCH)], sidx)
            pltpu.sync_copy(dst_hbm.at[pl.ds(scbase, SCH)], didx)
            for pc in range(NBUF - 1):
                start_gathers(pc, pc)

            def chunk_body(c, carry):
                buf = lax.rem(c, NBUF)
                wait_gathers(c, buf)

                @pl.when(c + NBUF - 1 < n_chunks)
                def _():
                    start_gathers(c + NBUF - 1, lax.rem(c + NBUF - 1, NBUF))

                rbase = buf * E

                def group_body(g, carry2):
                    lane = lax.iota(jnp.int32, L)
                    ebase = rbase + g * L
                    for l in range(L):
                        e = ebase + l
                        p = srows[e, pl.ds(0, L)] * drows[e, pl.ds(0, L)]
                        for k in range(1, D // L):
                            p = p + (srows[e, pl.ds(k * L, L)]
                                     * drows[e, pl.ds(k * L, L)])
                        pbuf[l, :] = p
                    acc = jnp.zeros((L,), jnp.float32)
                    cc = lane
                    for _ in range(L):
                        acc = acc + plsc.load_gather(pbuf, [lane, cc])
                        cc = (cc + 1) & (L - 1)
                    och[pl.ds(c * E + g * L, L)] = 1.0 / (1.0 + jnp.exp(-acc))
                    return carry2

                lax.fori_loop(0, E // L, group_body, 0)
                return carry

            lax.fori_loop(0, n_chunks, chunk_body, 0)
            pltpu.sync_copy(och, out_hbm.at[pl.ds(scbase, SCH)])
            return carry0

        lax.fori_loop(0, n_sch, sch_body, 0)

    return decode


def kernel(z, edge_index):
    N = z.shape[0]
    B = edge_index.shape[1]
    decode = _make_decoder(N, B)
    return decode(z, edge_index[0], edge_index[1])


# speedup vs baseline: 1.0536x; 1.0536x over previous
import functools

import jax
import jax.numpy as jnp
from jax import lax
from jax.experimental import pallas as pl
from jax.experimental.pallas import tpu as pltpu
from jax.experimental.pallas import tpu_sc as plsc

D = 128
L = 16
L2 = 32
E = 80
SCH = 2000
NBUF = 2


@functools.lru_cache(maxsize=None)
def _make_decoder(N, B):
    info = plsc.get_sparse_core_info()
    NC, NS = info.num_cores, info.num_subcores
    NW = NC * NS
    assert B % NW == 0
    per_w = B // NW
    assert per_w % SCH == 0 and SCH % E == 0
    n_sch = per_w // SCH
    n_chunks = SCH // E
    mesh = plsc.VectorSubcoreMesh(core_axis_name="c", subcore_axis_name="s")

    @functools.partial(
        pl.kernel,
        out_type=jax.ShapeDtypeStruct((B,), jnp.float32),
        mesh=mesh,
        compiler_params=pltpu.CompilerParams(needs_layout_passes=False, use_tc_tiling_on_sc=False),
        scratch_types=[
            pltpu.VMEM((SCH,), jnp.int32),
            pltpu.VMEM((SCH,), jnp.int32),
            pltpu.VMEM((NBUF * E, D), jnp.bfloat16),
            pltpu.VMEM((NBUF * E, D), jnp.bfloat16),
            pltpu.VMEM((SCH,), jnp.float32),
            pltpu.VMEM((L, L), jnp.float32),
            pltpu.SemaphoreType.DMA((NBUF,)),
            pltpu.SemaphoreType.DMA((NBUF,)),
        ],
    )
    def decode(zb_hbm, src_hbm, dst_hbm, out_hbm,
               sidx, didx, srows, drows, och, pbuf, sem_s, sem_d):
        wid = lax.axis_index("s") * NC + lax.axis_index("c")
        wbase = wid * per_w

        def start_gathers(c, buf):
            pltpu.async_copy(
                zb_hbm.at[sidx.at[pl.ds(c * E, E)]],
                srows.at[pl.ds(buf * E, E)], sem_s.at[buf])
            pltpu.async_copy(
                zb_hbm.at[didx.at[pl.ds(c * E, E)]],
                drows.at[pl.ds(buf * E, E)], sem_d.at[buf])

        def wait_gathers(c, buf):
            pltpu.make_async_copy(
                zb_hbm.at[sidx.at[pl.ds(c * E, E)]],
                srows.at[pl.ds(buf * E, E)], sem_s.at[buf]).wait()
            pltpu.make_async_copy(
                zb_hbm.at[didx.at[pl.ds(c * E, E)]],
                drows.at[pl.ds(buf * E, E)], sem_d.at[buf]).wait()

        def sch_body(j, carry0):
            scbase = wbase + j * SCH
            pltpu.sync_copy(src_hbm.at[pl.ds(scbase, SCH)], sidx)
            pltpu.sync_copy(dst_hbm.at[pl.ds(scbase, SCH)], didx)
            for pc in range(NBUF - 1):
                start_gathers(pc, pc)

            def chunk_body(c, carry):
                buf = lax.rem(c, NBUF)
                wait_gathers(c, buf)

                @pl.when(c + NBUF - 1 < n_chunks)
                def _():
                    start_gathers(c + NBUF - 1, lax.rem(c + NBUF - 1, NBUF))

                rbase = buf * E

                def group_body(g, carry2):
                    lane = lax.iota(jnp.int32, L)
                    ebase = rbase + g * L
                    for l in range(L):
                        e = ebase + l
                        p = None
                        for k in range(D // L2):
                            vs = srows[e, pl.ds(k * L2, L2)]
                            vd = drows[e, pl.ds(k * L2, L2)]
                            s0, s1 = plsc.unpack(
                                vs, format=plsc.PackFormat.INTERLEAVED)
                            d0, d1 = plsc.unpack(
                                vd, format=plsc.PackFormat.INTERLEAVED)
                            q = s0 * d0 + s1 * d1
                            p = q if p is None else p + q
                        pbuf[l, :] = p
                    acc = jnp.zeros((L,), jnp.float32)
                    cc = lane
                    for _ in range(L):
                        acc = acc + plsc.load_gather(pbuf, [lane, cc])
                        cc = (cc + 1) & (L - 1)
                    och[pl.ds(c * E + g * L, L)] = 1.0 / (1.0 + jnp.exp(-acc))
                    return carry2

                lax.fori_loop(0, E // L, group_body, 0)
                return carry

            lax.fori_loop(0, n_chunks, chunk_body, 0)
            pltpu.sync_copy(och, out_hbm.at[pl.ds(scbase, SCH)])
            return carry0

        lax.fori_loop(0, n_sch, sch_body, 0)

    return decode


def kernel(z, edge_index):
    N = z.shape[0]
    B = edge_index.shape[1]
    decode = _make_decoder(N, B)
    zb = z.astype(jnp.bfloat16)
    return decode(zb, edge_index[0], edge_index[1])
